# baseline (speedup 1.0000x reference)
import functools

import jax
import jax.numpy as jnp
from jax import lax
from jax.experimental import pallas as pl
from jax.experimental.pallas import tpu as pltpu
from jax.experimental.pallas import tpu_sc as plsc

NC = 2
NS = 16
NW = NC * NS
_CHUNK = 80
_NBUF = 2
_BLK = 1000


def _sc_mesh():
    return plsc.VectorSubcoreMesh(core_axis_name="c", subcore_axis_name="s")


def _rows_copy(N, s, srcfn, dstfn):
    rpt0 = (-(-N // NS) + 7) // 8 * 8
    last = N - (NS - 1) * rpt0

    @pl.when(s < NS - 1)
    def _():
        sl = pl.ds(s * rpt0, rpt0)
        pltpu.sync_copy(srcfn(sl), dstfn(sl))

    @pl.when(s == NS - 1)
    def _():
        sl = pl.ds((NS - 1) * rpt0, last)
        pltpu.sync_copy(srcfn(sl), dstfn(sl))


@functools.cache
def _make_deg(E, N):
    ept = E // NW
    nvec = ept // 16
    nchunk = N // 16
    R = 8

    @functools.partial(
        pl.kernel,
        out_type=jax.ShapeDtypeStruct((NW * N,), jnp.float32),
        mesh=_sc_mesh(),
        compiler_params=pltpu.CompilerParams(needs_layout_passes=False),
        scratch_types=[
            pltpu.VMEM((ept,), jnp.int32),
            pltpu.VMEM((R * N,), jnp.float32),
            pltpu.VMEM((N,), jnp.float32),
        ],
    )
    def deg_k(dst_hbm, zeros_hbm, out_hbm, dst_v, tab_v, red_v):
        c = lax.axis_index("c")
        s = lax.axis_index("s")
        w = c * NS + s
        pltpu.sync_copy(dst_hbm.at[pl.ds(w * ept, ept)], dst_v)
        pltpu.sync_copy(zeros_hbm, tab_v)
        lane = lax.iota(jnp.int32, 16)
        offs = (lane % R) * N
        lo = lane < R
        hi = lane >= R
        ones = jnp.ones((16,), jnp.float32)

        def count(j, carry):
            idx = dst_v[pl.ds(j * 16, 16)] + offs
            plsc.addupdate_scatter(tab_v, [idx], ones, mask=lo)
            plsc.addupdate_scatter(tab_v, [idx], ones, mask=hi)
            return carry

        lax.fori_loop(0, nvec, count, 0)

        def reduce(m, carry):
            acc = tab_v[pl.ds(m * 16, 16)]
            for r in range(1, R):
                acc = acc + tab_v[pl.ds(r * N + m * 16, 16)]
            red_v[pl.ds(m * 16, 16)] = acc
            return carry

        lax.fori_loop(0, nchunk, reduce, 0)
        pltpu.sync_copy(red_v, out_hbm.at[pl.ds(w * N, N)])

    return deg_k


@functools.cache
def _make_agg(E, N, dim):
    ept = E // NW
    steps = ept // _CHUNK
    nbuf = _NBUF

    @functools.partial(
        pl.kernel,
        out_type=jax.ShapeDtypeStruct((NC, N, dim), jnp.float32),
        mesh=_sc_mesh(),
        scratch_types=(
            [pltpu.VMEM((ept,), jnp.int32),
             pltpu.VMEM((steps, _CHUNK), jnp.int32)]
            + [pltpu.VMEM((_CHUNK, dim), jnp.float32)] * nbuf
            + [pltpu.VMEM_SHARED((N, dim), jnp.float32)]
            + [pltpu.SemaphoreType.DMA] * nbuf
        ),
    )
    def agg_k(src_hbm, dst_hbm, table_hbm, zeros_hbm, out_hbm,
              src_v, dst_v, *rest):
        rows = rest[:nbuf]
        acc_sh = rest[nbuf]
        sems = rest[nbuf + 1:]
        c = lax.axis_index("c")
        s = lax.axis_index("s")
        _rows_copy(N, s, lambda sl: zeros_hbm.at[sl], lambda sl: acc_sh.at[sl])
        w = c * NS + s
        ebase = w * ept
        pltpu.sync_copy(src_hbm.at[pl.ds(ebase, ept)], src_v)
        pltpu.sync_copy(dst_hbm.at[w], dst_v)
        plsc.subcore_barrier()

        def gather_start(k, b):
            pltpu.async_copy(
                table_hbm.at[src_v.at[pl.ds(k * _CHUNK, _CHUNK)]],
                rows[b], sems[b])

        def gather_wait(k, b):
            pltpu.make_async_copy(
                table_hbm.at[src_v.at[pl.ds(k * _CHUNK, _CHUNK)]],
                rows[b], sems[b]).wait()

        def scatter_start(k, b):
            pltpu.async_copy(rows[b], acc_sh.at[dst_v.at[k]], sems[b],
                             add=True)

        def scatter_wait(k, b):
            pltpu.make_async_copy(rows[b], acc_sh.at[dst_v.at[k]],
                                  sems[b]).wait()

        for i in range(nbuf - 1):
            gather_start(i, i)

        def group(t, carry):
            for j in range(nbuf):
                k = t * nbuf + j
                gather_wait(k, j)
                scatter_start(k, j)
                if j == 0:
                    @pl.when(t > 0)
                    def _():
                        scatter_wait(k - 1, nbuf - 1)
                else:
                    scatter_wait(k - 1, j - 1)

                @pl.when(k + nbuf - 1 < steps)
                def _():
                    gather_start(k + nbuf - 1, (j + nbuf - 1) % nbuf)
            return carry

        lax.fori_loop(0, steps // nbuf, group, 0)
        for k in range((steps // nbuf) * nbuf, steps):
            gather_wait(k, k % nbuf)
            scatter_start(k, k % nbuf)
            scatter_wait(k - 1, (k - 1) % nbuf)
        scatter_wait(steps - 1, (steps - 1) % nbuf)
        plsc.subcore_barrier()
        _rows_copy(N, s, lambda sl: acc_sh.at[sl], lambda sl: out_hbm.at[c, sl])

    return agg_k


def _dinv_of(degp_ref):
    deg = jnp.sum(degp_ref[...], axis=1, keepdims=True) + 1.0
    return lax.rsqrt(deg)


@functools.cache
def _make_scale(N, D):
    grid = N // _BLK

    def body(x_ref, degp_ref, xs_ref):
        xs_ref[...] = x_ref[...] * _dinv_of(degp_ref)

    return pl.pallas_call(
        body,
        grid=(grid,),
        in_specs=[
            pl.BlockSpec((_BLK, D), lambda i: (i, 0)),
            pl.BlockSpec((_BLK, NW), lambda i: (i, 0)),
        ],
        out_specs=pl.BlockSpec((_BLK, D), lambda i: (i, 0)),
        out_shape=jax.ShapeDtypeStruct((N, D), jnp.float32),
    )


@functools.cache
def _make_mid(N, D, H1, H2):
    grid = N // _BLK

    def body(aggp_ref, xs_ref, degp_ref, W1_ref, b1_ref, W2_ref, hs2_ref):
        dinv = _dinv_of(degp_ref)
        t1 = (aggp_ref[0] + aggp_ref[1] + xs_ref[...]) * dinv
        out1 = jnp.maximum(
            jnp.dot(t1, W1_ref[...], preferred_element_type=jnp.float32)
            + b1_ref[...], 0.0)
        hs2 = jnp.dot(
            out1, W2_ref[...], preferred_element_type=jnp.float32) * dinv
        hs2_ref[...] = jnp.concatenate(
            [hs2, jnp.zeros((_BLK, 128 - H2), jnp.float32)], axis=1)

    return pl.pallas_call(
        body,
        grid=(grid,),
        in_specs=[
            pl.BlockSpec((NC, _BLK, D), lambda i: (0, i, 0)),
            pl.BlockSpec((_BLK, D), lambda i: (i, 0)),
            pl.BlockSpec((_BLK, NW), lambda i: (i, 0)),
            pl.BlockSpec((D, H1), lambda i: (0, 0)),
            pl.BlockSpec((1, H1), lambda i: (0, 0)),
            pl.BlockSpec((H1, H2), lambda i: (0, 0)),
        ],
        out_specs=pl.BlockSpec((_BLK, 128), lambda i: (i, 0)),
        out_shape=jax.ShapeDtypeStruct((N, 128), jnp.float32),
    )


@functools.cache
def _make_final(N, H2, Z, G):
    grid = N // _BLK

    def body(aggp_ref, hs2_ref, degp_ref, b2_ref, batch_ref, Wmu_ref, bmu_ref,
             z_ref, ps, cnt):
        i = pl.program_id(0)

        @pl.when(i == 0)
        def _init():
            ps[...] = jnp.zeros_like(ps)
            cnt[...] = jnp.zeros_like(cnt)

        dinv = _dinv_of(degp_ref)
        out2 = jnp.maximum(
            (aggp_ref[0, :, :H2] + aggp_ref[1, :, :H2] + hs2_ref[:, :H2])
            * dinv + b2_ref[...],
            0.0)
        b = batch_ref[0, 0, :]
        gids = lax.broadcasted_iota(jnp.int32, (G, _BLK), 0)
        M = (b[None, :] == gids).astype(jnp.float32)
        ps[...] += jnp.dot(M, out2, preferred_element_type=jnp.float32)
        csum = jnp.sum(M, axis=1, keepdims=True)
        cnt[...] += jnp.broadcast_to(csum, cnt.shape)

        @pl.when(i == grid - 1)
        def _fin():
            pooled = ps[...] / jnp.maximum(cnt[:, 0:1], 1.0)
            z_ref[...] = jnp.dot(
                pooled, Wmu_ref[...], preferred_element_type=jnp.float32
            ) + bmu_ref[...]

    return pl.pallas_call(
        body,
        grid=(grid,),
        in_specs=[
            pl.BlockSpec((NC, _BLK, 128), lambda i: (0, i, 0)),
            pl.BlockSpec((_BLK, 128), lambda i: (i, 0)),
            pl.BlockSpec((_BLK, NW), lambda i: (i, 0)),
            pl.BlockSpec((1, H2), lambda i: (0, 0)),
            pl.BlockSpec((1, 1, _BLK), lambda i: (i, 0, 0)),
            pl.BlockSpec((H2, Z), lambda i: (0, 0)),
            pl.BlockSpec((1, Z), lambda i: (0, 0)),
        ],
        out_specs=pl.BlockSpec((G, Z), lambda i: (0, 0)),
        out_shape=jax.ShapeDtypeStruct((G, Z), jnp.float32),
        scratch_shapes=[
            pltpu.VMEM((G, H2), jnp.float32),
            pltpu.VMEM((G, 128), jnp.float32),
        ],
    )


def kernel(x, edge_index, batch, W1, b1, W2, b2, Wmu, bmu):
    N, D = x.shape
    E = edge_index.shape[1]
    H1 = W1.shape[1]
    H2 = W2.shape[1]
    Z = Wmu.shape[1]
    G = 64

    src = edge_index[0]
    dst = edge_index[1]

    degp = _make_deg(E, N)(
        dst, jnp.zeros((8 * N,), jnp.float32)).reshape(NW, N).T
    xs = _make_scale(N, D)(x, degp)
    ept = E // NW
    dst3 = dst.reshape(NW, ept // _CHUNK, _CHUNK)
    agg1 = _make_agg(E, N, D)(src, dst3, xs, jnp.zeros((N, D), jnp.float32))
    hs2 = _make_mid(N, D, H1, H2)(
        agg1, xs, degp, W1, b1.reshape(1, H1), W2)
    agg2 = _make_agg(E, N, 128)(src, dst3, hs2, jnp.zeros((N, 128), jnp.float32))
    z = _make_final(N, H2, Z, G)(
        agg2, hs2, degp, b2.reshape(1, H2),
        batch.reshape(N // _BLK, 1, _BLK), Wmu, bmu.reshape(1, Z))
    return z


# speedup vs baseline: 1.3797x; 1.3797x over previous
import functools

import jax
import jax.numpy as jnp
from jax import lax
from jax.experimental import pallas as pl
from jax.experimental.pallas import tpu as pltpu
from jax.experimental.pallas import tpu_sc as plsc

NC = 2
NS = 16
NW = NC * NS
_CHUNK = 80
_NBUF = 3
_BLK = 1000


def _sc_mesh():
    return plsc.VectorSubcoreMesh(core_axis_name="c", subcore_axis_name="s")


def _rows_copy(N, s, srcfn, dstfn):
    rpt0 = (-(-N // NS) + 7) // 8 * 8
    last = N - (NS - 1) * rpt0

    @pl.when(s < NS - 1)
    def _():
        sl = pl.ds(s * rpt0, rpt0)
        pltpu.sync_copy(srcfn(sl), dstfn(sl))

    @pl.when(s == NS - 1)
    def _():
        sl = pl.ds((NS - 1) * rpt0, last)
        pltpu.sync_copy(srcfn(sl), dstfn(sl))


@functools.cache
def _make_deg(E, N):
    ept = E // NW
    nvec = ept // 16
    nchunk = N // 16
    R = 8

    @functools.partial(
        pl.kernel,
        out_type=jax.ShapeDtypeStruct((NW * N,), jnp.float32),
        mesh=_sc_mesh(),
        compiler_params=pltpu.CompilerParams(needs_layout_passes=False),
        scratch_types=[
            pltpu.VMEM((ept,), jnp.int32),
            pltpu.VMEM((R * N,), jnp.float32),
            pltpu.VMEM((N,), jnp.float32),
        ],
    )
    def deg_k(dst_hbm, zeros_hbm, out_hbm, dst_v, tab_v, red_v):
        c = lax.axis_index("c")
        s = lax.axis_index("s")
        w = c * NS + s
        pltpu.sync_copy(dst_hbm.at[pl.ds(w * ept, ept)], dst_v)
        pltpu.sync_copy(zeros_hbm, tab_v)
        lane = lax.iota(jnp.int32, 16)
        offs = (lane % R) * N
        lo = lane < R
        hi = lane >= R
        ones = jnp.ones((16,), jnp.float32)

        def count(j, carry):
            idx = dst_v[pl.ds(j * 16, 16)] + offs
            plsc.addupdate_scatter(tab_v, [idx], ones, mask=lo)
            plsc.addupdate_scatter(tab_v, [idx], ones, mask=hi)
            return carry

        lax.fori_loop(0, nvec, count, 0)

        def reduce(m, carry):
            acc = tab_v[pl.ds(m * 16, 16)]
            for r in range(1, R):
                acc = acc + tab_v[pl.ds(r * N + m * 16, 16)]
            red_v[pl.ds(m * 16, 16)] = acc
            return carry

        lax.fori_loop(0, nchunk, reduce, 0)
        pltpu.sync_copy(red_v, out_hbm.at[pl.ds(w * N, N)])

    return deg_k


@functools.cache
def _make_agg(E, N, dim):
    ept = E // NW
    steps = ept // _CHUNK
    nbuf = _NBUF

    @functools.partial(
        pl.kernel,
        out_type=jax.ShapeDtypeStruct((NC, N, dim), jnp.float32),
        mesh=_sc_mesh(),
        scratch_types=(
            [pltpu.VMEM((ept,), jnp.int32),
             pltpu.VMEM((ept,), jnp.int32)]
            + [pltpu.VMEM((_CHUNK, dim), jnp.float32)] * nbuf
            + [pltpu.VMEM_SHARED((N, dim), jnp.float32)]
            + [pltpu.SemaphoreType.DMA] * nbuf
        ),
    )
    def agg_k(src_hbm, dst_hbm, table_hbm, zeros_hbm, out_hbm,
              src_v, dst_v, *rest):
        rows = rest[:nbuf]
        acc_sh = rest[nbuf]
        sems = rest[nbuf + 1:]
        c = lax.axis_index("c")
        s = lax.axis_index("s")
        _rows_copy(N, s, lambda sl: zeros_hbm.at[sl], lambda sl: acc_sh.at[sl])
        w = c * NS + s
        ebase = w * ept
        pltpu.sync_copy(src_hbm.at[pl.ds(ebase, ept)], src_v)
        pltpu.sync_copy(dst_hbm.at[pl.ds(ebase, ept)], dst_v)
        plsc.subcore_barrier()

        def gather_start(k, b):
            pltpu.async_copy(
                table_hbm.at[src_v.at[pl.ds(k * _CHUNK, _CHUNK)]],
                rows[b], sems[b])

        def gather_wait(k, b):
            pltpu.make_async_copy(
                table_hbm.at[src_v.at[pl.ds(k * _CHUNK, _CHUNK)]],
                rows[b], sems[b]).wait()

        def scatter_start(k, b):
            pltpu.async_copy(
                rows[b], acc_sh.at[dst_v.at[pl.ds(k * _CHUNK, _CHUNK)]],
                sems[b], add=True)

        def scatter_wait(k, b):
            pltpu.make_async_copy(
                rows[b], acc_sh.at[dst_v.at[pl.ds(k * _CHUNK, _CHUNK)]],
                sems[b]).wait()

        for i in range(nbuf - 1):
            gather_start(i, i)

        def group(t, carry):
            for j in range(nbuf):
                k = t * nbuf + j
                gather_wait(k, j)
                scatter_start(k, j)
                if j == 0:
                    @pl.when(t > 0)
                    def _():
                        scatter_wait(k - 1, nbuf - 1)
                else:
                    scatter_wait(k - 1, j - 1)

                @pl.when(k + nbuf - 1 < steps)
                def _():
                    gather_start(k + nbuf - 1, (j + nbuf - 1) % nbuf)
            return carry

        lax.fori_loop(0, steps // nbuf, group, 0)
        for k in range((steps // nbuf) * nbuf, steps):
            gather_wait(k, k % nbuf)
            scatter_start(k, k % nbuf)
            scatter_wait(k - 1, (k - 1) % nbuf)
        scatter_wait(steps - 1, (steps - 1) % nbuf)
        plsc.subcore_barrier()
        _rows_copy(N, s, lambda sl: acc_sh.at[sl], lambda sl: out_hbm.at[c, sl])

    return agg_k


def _dinv_of(degp_ref):
    deg = jnp.sum(degp_ref[...], axis=1, keepdims=True) + 1.0
    return lax.rsqrt(deg)


@functools.cache
def _make_scale(N, D):
    grid = N // _BLK

    def body(x_ref, degp_ref, xs_ref):
        xs_ref[...] = x_ref[...] * _dinv_of(degp_ref)

    return pl.pallas_call(
        body,
        grid=(grid,),
        in_specs=[
            pl.BlockSpec((_BLK, D), lambda i: (i, 0)),
            pl.BlockSpec((_BLK, NW), lambda i: (i, 0)),
        ],
        out_specs=pl.BlockSpec((_BLK, D), lambda i: (i, 0)),
        out_shape=jax.ShapeDtypeStruct((N, D), jnp.float32),
    )


@functools.cache
def _make_mid(N, D, H1, H2):
    grid = N // _BLK

    def body(aggp_ref, xs_ref, degp_ref, W1_ref, b1_ref, W2_ref, hs2_ref):
        dinv = _dinv_of(degp_ref)
        t1 = (aggp_ref[0] + aggp_ref[1] + xs_ref[...]) * dinv
        out1 = jnp.maximum(
            jnp.dot(t1, W1_ref[...], preferred_element_type=jnp.float32)
            + b1_ref[...], 0.0)
        hs2 = jnp.dot(
            out1, W2_ref[...], preferred_element_type=jnp.float32) * dinv
        hs2_ref[...] = jnp.concatenate(
            [hs2, jnp.zeros((_BLK, 128 - H2), jnp.float32)], axis=1)

    return pl.pallas_call(
        body,
        grid=(grid,),
        in_specs=[
            pl.BlockSpec((NC, _BLK, D), lambda i: (0, i, 0)),
            pl.BlockSpec((_BLK, D), lambda i: (i, 0)),
            pl.BlockSpec((_BLK, NW), lambda i: (i, 0)),
            pl.BlockSpec((D, H1), lambda i: (0, 0)),
            pl.BlockSpec((1, H1), lambda i: (0, 0)),
            pl.BlockSpec((H1, H2), lambda i: (0, 0)),
        ],
        out_specs=pl.BlockSpec((_BLK, 128), lambda i: (i, 0)),
        out_shape=jax.ShapeDtypeStruct((N, 128), jnp.float32),
    )


@functools.cache
def _make_final(N, H2, Z, G):
    grid = N // _BLK

    def body(aggp_ref, hs2_ref, degp_ref, b2_ref, batch_ref, Wmu_ref, bmu_ref,
             z_ref, ps, cnt):
        i = pl.program_id(0)

        @pl.when(i == 0)
        def _init():
            ps[...] = jnp.zeros_like(ps)
            cnt[...] = jnp.zeros_like(cnt)

        dinv = _dinv_of(degp_ref)
        out2 = jnp.maximum(
            (aggp_ref[0, :, :H2] + aggp_ref[1, :, :H2] + hs2_ref[:, :H2])
            * dinv + b2_ref[...],
            0.0)
        b = batch_ref[0, 0, :]
        gids = lax.broadcasted_iota(jnp.int32, (G, _BLK), 0)
        M = (b[None, :] == gids).astype(jnp.float32)
        ps[...] += jnp.dot(M, out2, preferred_element_type=jnp.float32)
        csum = jnp.sum(M, axis=1, keepdims=True)
        cnt[...] += jnp.broadcast_to(csum, cnt.shape)

        @pl.when(i == grid - 1)
        def _fin():
            pooled = ps[...] / jnp.maximum(cnt[:, 0:1], 1.0)
            z_ref[...] = jnp.dot(
                pooled, Wmu_ref[...], preferred_element_type=jnp.float32
            ) + bmu_ref[...]

    return pl.pallas_call(
        body,
        grid=(grid,),
        in_specs=[
            pl.BlockSpec((NC, _BLK, 128), lambda i: (0, i, 0)),
            pl.BlockSpec((_BLK, 128), lambda i: (i, 0)),
            pl.BlockSpec((_BLK, NW), lambda i: (i, 0)),
            pl.BlockSpec((1, H2), lambda i: (0, 0)),
            pl.BlockSpec((1, 1, _BLK), lambda i: (i, 0, 0)),
            pl.BlockSpec((H2, Z), lambda i: (0, 0)),
            pl.BlockSpec((1, Z), lambda i: (0, 0)),
        ],
        out_specs=pl.BlockSpec((G, Z), lambda i: (0, 0)),
        out_shape=jax.ShapeDtypeStruct((G, Z), jnp.float32),
        scratch_shapes=[
            pltpu.VMEM((G, H2), jnp.float32),
            pltpu.VMEM((G, H2), jnp.float32),
        ],
    )


def kernel(x, edge_index, batch, W1, b1, W2, b2, Wmu, bmu):
    N, D = x.shape
    E = edge_index.shape[1]
    H1 = W1.shape[1]
    H2 = W2.shape[1]
    Z = Wmu.shape[1]
    G = 64

    src = edge_index[0]
    dst = edge_index[1]

    degp = _make_deg(E, N)(
        dst, jnp.zeros((8 * N,), jnp.float32)).reshape(NW, N).T
    xs = _make_scale(N, D)(x, degp)
    agg1 = _make_agg(E, N, D)(src, dst, xs, jnp.zeros((N, D), jnp.float32))
    hs2 = _make_mid(N, D, H1, H2)(
        agg1, xs, degp, W1, b1.reshape(1, H1), W2)
    agg2 = _make_agg(E, N, 128)(src, dst, hs2, jnp.zeros((N, 128), jnp.float32))
    z = _make_final(N, H2, Z, G)(
        agg2, hs2, degp, b2.reshape(1, H2),
        batch.reshape(N // _BLK, 1, _BLK), Wmu, bmu.reshape(1, Z))
    return z


# speedup vs baseline: 1.4399x; 1.0436x over previous
import functools

import jax
import jax.numpy as jnp
from jax import lax
from jax.experimental import pallas as pl
from jax.experimental.pallas import tpu as pltpu
from jax.experimental.pallas import tpu_sc as plsc

NC = 2
NS = 16
NW = NC * NS
_CHUNK = 40
_NBUF = 6
_BLK = 1000


def _sc_mesh():
    return plsc.VectorSubcoreMesh(core_axis_name="c", subcore_axis_name="s")


def _rows_copy(N, s, srcfn, dstfn):
    rpt0 = (-(-N // NS) + 7) // 8 * 8
    last = N - (NS - 1) * rpt0

    @pl.when(s < NS - 1)
    def _():
        sl = pl.ds(s * rpt0, rpt0)
        pltpu.sync_copy(srcfn(sl), dstfn(sl))

    @pl.when(s == NS - 1)
    def _():
        sl = pl.ds((NS - 1) * rpt0, last)
        pltpu.sync_copy(srcfn(sl), dstfn(sl))


@functools.cache
def _make_deg(E, N):
    ept = E // NW
    nvec = ept // 16
    nchunk = N // 16
    R = 8

    @functools.partial(
        pl.kernel,
        out_type=jax.ShapeDtypeStruct((NW * N,), jnp.float32),
        mesh=_sc_mesh(),
        compiler_params=pltpu.CompilerParams(needs_layout_passes=False),
        scratch_types=[
            pltpu.VMEM((ept,), jnp.int32),
            pltpu.VMEM((R * N,), jnp.float32),
            pltpu.VMEM((N,), jnp.float32),
        ],
    )
    def deg_k(dst_hbm, zeros_hbm, out_hbm, dst_v, tab_v, red_v):
        c = lax.axis_index("c")
        s = lax.axis_index("s")
        w = c * NS + s
        pltpu.sync_copy(dst_hbm.at[pl.ds(w * ept, ept)], dst_v)
        pltpu.sync_copy(zeros_hbm, tab_v)
        lane = lax.iota(jnp.int32, 16)
        offs = (lane % R) * N
        lo = lane < R
        hi = lane >= R
        ones = jnp.ones((16,), jnp.float32)

        def count(j, carry):
            idx = dst_v[pl.ds(j * 16, 16)] + offs
            plsc.addupdate_scatter(tab_v, [idx], ones, mask=lo)
            plsc.addupdate_scatter(tab_v, [idx], ones, mask=hi)
            return carry

        lax.fori_loop(0, nvec, count, 0)

        def reduce(m, carry):
            acc = tab_v[pl.ds(m * 16, 16)]
            for r in range(1, R):
                acc = acc + tab_v[pl.ds(r * N + m * 16, 16)]
            red_v[pl.ds(m * 16, 16)] = acc
            return carry

        lax.fori_loop(0, nchunk, reduce, 0)
        pltpu.sync_copy(red_v, out_hbm.at[pl.ds(w * N, N)])

    return deg_k


@functools.cache
def _make_agg(E, N, dim):
    ept = E // NW
    steps = ept // _CHUNK
    nbuf = _NBUF

    @functools.partial(
        pl.kernel,
        out_type=jax.ShapeDtypeStruct((NC, N, dim), jnp.float32),
        mesh=_sc_mesh(),
        scratch_types=(
            [pltpu.VMEM((ept,), jnp.int32),
             pltpu.VMEM((ept,), jnp.int32)]
            + [pltpu.VMEM((_CHUNK, dim), jnp.float32)] * nbuf
            + [pltpu.VMEM_SHARED((N, dim), jnp.float32)]
            + [pltpu.SemaphoreType.DMA] * nbuf
        ),
    )
    def agg_k(src_hbm, dst_hbm, table_hbm, zeros_hbm, out_hbm,
              src_v, dst_v, *rest):
        rows = rest[:nbuf]
        acc_sh = rest[nbuf]
        sems = rest[nbuf + 1:]
        c = lax.axis_index("c")
        s = lax.axis_index("s")
        _rows_copy(N, s, lambda sl: zeros_hbm.at[sl], lambda sl: acc_sh.at[sl])
        w = c * NS + s
        ebase = w * ept
        pltpu.sync_copy(src_hbm.at[pl.ds(ebase, ept)], src_v)
        pltpu.sync_copy(dst_hbm.at[pl.ds(ebase, ept)], dst_v)
        plsc.subcore_barrier()

        def gather_start(k, b):
            pltpu.async_copy(
                table_hbm.at[src_v.at[pl.ds(k * _CHUNK, _CHUNK)]],
                rows[b], sems[b])

        def gather_wait(k, b):
            pltpu.make_async_copy(
                table_hbm.at[src_v.at[pl.ds(k * _CHUNK, _CHUNK)]],
                rows[b], sems[b]).wait()

        def scatter_start(k, b):
            pltpu.async_copy(
                rows[b], acc_sh.at[dst_v.at[pl.ds(k * _CHUNK, _CHUNK)]],
                sems[b], add=True)

        def scatter_wait(k, b):
            pltpu.make_async_copy(
                rows[b], acc_sh.at[dst_v.at[pl.ds(k * _CHUNK, _CHUNK)]],
                sems[b]).wait()

        for i in range(nbuf - 1):
            gather_start(i, i)

        def group(t, carry):
            for j in range(nbuf):
                k = t * nbuf + j
                gather_wait(k, j)
                scatter_start(k, j)
                if j == 0:
                    @pl.when(t > 0)
                    def _():
                        scatter_wait(k - 1, nbuf - 1)
                else:
                    scatter_wait(k - 1, j - 1)

                @pl.when(k + nbuf - 1 < steps)
                def _():
                    gather_start(k + nbuf - 1, (j + nbuf - 1) % nbuf)
            return carry

        lax.fori_loop(0, steps // nbuf, group, 0)
        for k in range((steps // nbuf) * nbuf, steps):
            gather_wait(k, k % nbuf)
            scatter_start(k, k % nbuf)
            scatter_wait(k - 1, (k - 1) % nbuf)
        scatter_wait(steps - 1, (steps - 1) % nbuf)
        plsc.subcore_barrier()
        _rows_copy(N, s, lambda sl: acc_sh.at[sl], lambda sl: out_hbm.at[c, sl])

    return agg_k


def _dinv_of(degp_ref):
    deg = jnp.sum(degp_ref[...], axis=1, keepdims=True) + 1.0
    return lax.rsqrt(deg)


@functools.cache
def _make_scale(N, D):
    grid = N // _BLK

    def body(x_ref, degp_ref, xs_ref):
        xs_ref[...] = x_ref[...] * _dinv_of(degp_ref)

    return pl.pallas_call(
        body,
        grid=(grid,),
        in_specs=[
            pl.BlockSpec((_BLK, D), lambda i: (i, 0)),
            pl.BlockSpec((_BLK, NW), lambda i: (i, 0)),
        ],
        out_specs=pl.BlockSpec((_BLK, D), lambda i: (i, 0)),
        out_shape=jax.ShapeDtypeStruct((N, D), jnp.float32),
    )


@functools.cache
def _make_mid(N, D, H1, H2):
    grid = N // _BLK

    def body(aggp_ref, xs_ref, degp_ref, W1_ref, b1_ref, W2_ref, hs2_ref):
        dinv = _dinv_of(degp_ref)
        t1 = (aggp_ref[0] + aggp_ref[1] + xs_ref[...]) * dinv
        out1 = jnp.maximum(
            jnp.dot(t1, W1_ref[...], preferred_element_type=jnp.float32)
            + b1_ref[...], 0.0)
        hs2 = jnp.dot(
            out1, W2_ref[...], preferred_element_type=jnp.float32) * dinv
        hs2_ref[...] = jnp.concatenate(
            [hs2, jnp.zeros((_BLK, 128 - H2), jnp.float32)], axis=1)

    return pl.pallas_call(
        body,
        grid=(grid,),
        in_specs=[
            pl.BlockSpec((NC, _BLK, D), lambda i: (0, i, 0)),
            pl.BlockSpec((_BLK, D), lambda i: (i, 0)),
            pl.BlockSpec((_BLK, NW), lambda i: (i, 0)),
            pl.BlockSpec((D, H1), lambda i: (0, 0)),
            pl.BlockSpec((1, H1), lambda i: (0, 0)),
            pl.BlockSpec((H1, H2), lambda i: (0, 0)),
        ],
        out_specs=pl.BlockSpec((_BLK, 128), lambda i: (i, 0)),
        out_shape=jax.ShapeDtypeStruct((N, 128), jnp.float32),
    )


@functools.cache
def _make_final(N, H2, Z, G):
    grid = N // _BLK

    def body(aggp_ref, hs2_ref, degp_ref, b2_ref, batch_ref, Wmu_ref, bmu_ref,
             z_ref, ps, cnt):
        i = pl.program_id(0)

        @pl.when(i == 0)
        def _init():
            ps[...] = jnp.zeros_like(ps)
            cnt[...] = jnp.zeros_like(cnt)

        dinv = _dinv_of(degp_ref)
        out2 = jnp.maximum(
            (aggp_ref[0, :, :H2] + aggp_ref[1, :, :H2] + hs2_ref[:, :H2])
            * dinv + b2_ref[...],
            0.0)
        b = batch_ref[0, 0, :]
        gids = lax.broadcasted_iota(jnp.int32, (G, _BLK), 0)
        M = (b[None, :] == gids).astype(jnp.float32)
        ps[...] += jnp.dot(M, out2, preferred_element_type=jnp.float32)
        csum = jnp.sum(M, axis=1, keepdims=True)
        cnt[...] += jnp.broadcast_to(csum, cnt.shape)

        @pl.when(i == grid - 1)
        def _fin():
            pooled = ps[...] / jnp.maximum(cnt[:, 0:1], 1.0)
            z_ref[...] = jnp.dot(
                pooled, Wmu_ref[...], preferred_element_type=jnp.float32
            ) + bmu_ref[...]

    return pl.pallas_call(
        body,
        grid=(grid,),
        in_specs=[
            pl.BlockSpec((NC, _BLK, 128), lambda i: (0, i, 0)),
            pl.BlockSpec((_BLK, 128), lambda i: (i, 0)),
            pl.BlockSpec((_BLK, NW), lambda i: (i, 0)),
            pl.BlockSpec((1, H2), lambda i: (0, 0)),
            pl.BlockSpec((1, 1, _BLK), lambda i: (i, 0, 0)),
            pl.BlockSpec((H2, Z), lambda i: (0, 0)),
            pl.BlockSpec((1, Z), lambda i: (0, 0)),
        ],
        out_specs=pl.BlockSpec((G, Z), lambda i: (0, 0)),
        out_shape=jax.ShapeDtypeStruct((G, Z), jnp.float32),
        scratch_shapes=[
            pltpu.VMEM((G, H2), jnp.float32),
            pltpu.VMEM((G, H2), jnp.float32),
        ],
    )


def kernel(x, edge_index, batch, W1, b1, W2, b2, Wmu, bmu):
    N, D = x.shape
    E = edge_index.shape[1]
    H1 = W1.shape[1]
    H2 = W2.shape[1]
    Z = Wmu.shape[1]
    G = 64

    src = edge_index[0]
    dst = edge_index[1]

    degp = _make_deg(E, N)(
        dst, jnp.zeros((8 * N,), jnp.float32)).reshape(NW, N).T
    xs = _make_scale(N, D)(x, degp)
    agg1 = _make_agg(E, N, D)(src, dst, xs, jnp.zeros((N, D), jnp.float32))
    hs2 = _make_mid(N, D, H1, H2)(
        agg1, xs, degp, W1, b1.reshape(1, H1), W2)
    agg2 = _make_agg(E, N, 128)(src, dst, hs2, jnp.zeros((N, 128), jnp.float32))
    z = _make_final(N, H2, Z, G)(
        agg2, hs2, degp, b2.reshape(1, H2),
        batch.reshape(N // _BLK, 1, _BLK), Wmu, bmu.reshape(1, Z))
    return z
